# baseline (speedup 1.0000x reference)
import jax
import jax.numpy as jnp
from jax.experimental import pallas as pl
from jax.experimental.pallas import tpu as pltpu

MAX_POS = 8192
EMBED_DIM = 1024
SEQ_LEN = 8192

_N_CHUNKS = 8
_CHUNK_ROWS = SEQ_LEN // _N_CHUNKS


def _dma_copy(pe_ref, out_ref, sems):
    for i in range(_N_CHUNKS):
        pltpu.make_async_copy(
            pe_ref.at[pl.ds(i * _CHUNK_ROWS, _CHUNK_ROWS), :],
            out_ref.at[pl.ds(i * _CHUNK_ROWS, _CHUNK_ROWS), :],
            sems.at[i],
        ).start()
    for i in range(_N_CHUNKS):
        pltpu.make_async_copy(
            pe_ref.at[pl.ds(i * _CHUNK_ROWS, _CHUNK_ROWS), :],
            out_ref.at[pl.ds(i * _CHUNK_ROWS, _CHUNK_ROWS), :],
            sems.at[i],
        ).wait()


def kernel(x, pe):
    out = pl.pallas_call(
        _dma_copy,
        in_specs=[pl.BlockSpec(memory_space=pl.ANY)],
        out_specs=pl.BlockSpec(memory_space=pl.ANY),
        out_shape=jax.ShapeDtypeStruct((SEQ_LEN, EMBED_DIM), pe.dtype),
        scratch_shapes=[pltpu.SemaphoreType.DMA((_N_CHUNKS,))],
    )(pe)
    return out[None]


# speedup vs baseline: 41.4479x; 41.4479x over previous
import jax
import jax.numpy as jnp
from jax.experimental import pallas as pl

MAX_POS = 8192
EMBED_DIM = 1024
SEQ_LEN = 8192

_BLOCK_ROWS = 512


def _copy_block(pe_ref, out_ref):
    out_ref[...] = pe_ref[...]


def kernel(x, pe):
    out = pl.pallas_call(
        _copy_block,
        grid=(MAX_POS // _BLOCK_ROWS,),
        in_specs=[pl.BlockSpec((_BLOCK_ROWS, EMBED_DIM), lambda i: (i, 0))],
        out_specs=pl.BlockSpec((_BLOCK_ROWS, EMBED_DIM), lambda i: (i, 0)),
        out_shape=jax.ShapeDtypeStruct((SEQ_LEN, EMBED_DIM), pe.dtype),
    )(pe)
    return out[None]


# speedup vs baseline: 49.1139x; 1.1850x over previous
import jax
import jax.numpy as jnp
from jax.experimental import pallas as pl

MAX_POS = 8192
EMBED_DIM = 1024
SEQ_LEN = 8192

_BLOCK_ROWS = 2048


def _copy_block(pe_ref, out_ref):
    out_ref[...] = pe_ref[...]


def kernel(x, pe):
    out = pl.pallas_call(
        _copy_block,
        grid=(MAX_POS // _BLOCK_ROWS,),
        in_specs=[pl.BlockSpec((_BLOCK_ROWS, EMBED_DIM), lambda i: (i, 0))],
        out_specs=pl.BlockSpec((_BLOCK_ROWS, EMBED_DIM), lambda i: (i, 0)),
        out_shape=jax.ShapeDtypeStruct((SEQ_LEN, EMBED_DIM), pe.dtype),
    )(pe)
    return out[None]
